# Initial kernel scaffold; baseline (speedup 1.0000x reference)
import jax
import jax.numpy as jnp
from jax.experimental import pallas as pl


def kernel(x, pos, Wr1, br1, Wr2, br2, Wr3, br3, Wf1, bf1, g1, be1, Wf2, bf2, g2, be2):
    raise NotImplementedError("write your pallas kernel here")


# speedup vs baseline: 14.4668x; 14.4668x over previous
import functools

import jax
import jax.numpy as jnp
from jax import lax
from jax.experimental import pallas as pl
from jax.experimental.pallas import tpu as pltpu
from jax.experimental.pallas import tpu_sc as plsc

BZ = 8
N = 2048
K = 16
RB = 256
NBLK = BZ * (N // RB)
NW = 32
ROWS = BZ * N * K
RPW = ROWS // NW
CHUNK = 128
NCH = RPW // CHUNK


def _topk_body(pos_ref, post_ref, idx_ref):
    b = pl.program_id(0)
    p = pos_ref[0]
    pt = post_ref[0]
    sq_r = jnp.sum(p * p, axis=1, keepdims=True)
    sq_c = jnp.sum(pt * pt, axis=0, keepdims=True)
    d = sq_r + sq_c - 2.0 * jnp.dot(p, pt, preferred_element_type=jnp.float32)
    iota = lax.broadcasted_iota(jnp.int32, (RB, N), 1)
    lane = lax.broadcasted_iota(jnp.int32, (RB, K), 1)
    out = jnp.zeros((RB, K), jnp.int32)
    for r in range(K):
        m = jnp.min(d, axis=1, keepdims=True)
        eq = d == m
        sel = jnp.min(jnp.where(eq, iota, N), axis=1, keepdims=True)
        d = jnp.where(iota == sel, jnp.float32(jnp.inf), d)
        out = jnp.where(lane == r, sel, out)
    idx_ref[0] = out + b * N


def _topk(pos, post):
    return pl.pallas_call(
        _topk_body,
        grid=(BZ, N // RB),
        in_specs=[
            pl.BlockSpec((1, RB, 3), lambda b, n: (b, n, 0)),
            pl.BlockSpec((1, 3, N), lambda b, n: (b, 0, 0)),
        ],
        out_specs=pl.BlockSpec((1, RB, K), lambda b, n: (b, n, 0)),
        out_shape=jax.ShapeDtypeStruct((BZ, N, K), jnp.int32),
    )(pos, post)


def _sc_gather_body(table, gidx, out, idx_v, rows_v, sem):
    c = lax.axis_index("c")
    s = lax.axis_index("s")
    wid = s * 2 + c
    pltpu.sync_copy(gidx.at[wid], idx_v)

    def chunk(j, carry):
        pltpu.async_copy(table.at[idx_v.at[j]], rows_v, sem).wait()
        pltpu.sync_copy(rows_v, out.at[pl.ds(wid * RPW + j * CHUNK, CHUNK)])
        return carry

    lax.fori_loop(0, NCH, chunk, 0)


@functools.lru_cache(maxsize=1)
def _sc_gather_kernel():
    return pl.kernel(
        _sc_gather_body,
        out_type=jax.ShapeDtypeStruct((ROWS, 128), jnp.float32),
        mesh=plsc.VectorSubcoreMesh(core_axis_name="c", subcore_axis_name="s"),
        scratch_types=[
            pltpu.VMEM((NCH, CHUNK), jnp.int32),
            pltpu.VMEM((CHUNK, 128), jnp.float32),
            pltpu.SemaphoreType.DMA,
        ],
    )


def _sc_gather(table, gidx):
    return _sc_gather_kernel()(table, gidx)


def _relmlp_body(g_ref, pos_ref, wa_ref, wb_ref, wc_ref, w9_ref, br1_ref,
                 wr2_ref, br2_ref, wr3_ref, br3_ref, wf1_ref, bf1_ref,
                 h1_ref, acc_ref):
    i = pl.program_id(0)
    bf16 = jnp.bfloat16
    dot = functools.partial(jnp.dot, preferred_element_type=jnp.float32)
    g = g_ref[0]
    p = pos_ref[0]
    gpos = g[:, 0:3]
    center = jnp.broadcast_to(p[:, None, :], (RB, K, 3)).reshape(RB * K, 3)
    rel = gpos - center
    euclid = jnp.sqrt(jnp.sum(rel * rel, axis=1, keepdims=True) + 1e-12)
    pc = dot(p.astype(bf16), wa_ref[...].astype(bf16))
    pc = jnp.broadcast_to(pc[:, None, :], (RB, K, 32)).reshape(RB * K, 32)
    hg = dot(gpos.astype(bf16), wb_ref[...].astype(bf16))
    hr = dot(rel.astype(bf16), wc_ref[...].astype(bf16))
    he = (euclid.astype(bf16).astype(jnp.float32)
          * w9_ref[...].astype(bf16).astype(jnp.float32))
    h = jax.nn.relu(pc + hg + hr + he + br1_ref[...])
    h = jax.nn.relu(
        dot(h.astype(bf16), wr2_ref[...].astype(bf16)) + br2_ref[...])
    cw = dot(h.astype(bf16), wr3_ref[...].astype(bf16)) \
        + br3_ref[...]
    prod = cw * g
    gf = jax.nn.relu(jnp.max(prod.reshape(RB, K, 128), axis=1))
    h1 = dot(gf.astype(bf16), wf1_ref[...].astype(bf16)) \
        + bf1_ref[...]
    h1_ref[0] = h1

    @pl.when(i == 0)
    def _():
        acc_ref[...] = jnp.zeros((8, 128), jnp.float32)

    acc_ref[0:1, :] += jnp.sum(h1, axis=0, keepdims=True)
    acc_ref[1:2, :] += jnp.sum(h1 * h1, axis=0, keepdims=True)


def _relmlp(g, pos, wa, wb, wc, w9, br1, wr2, br2, wr3p, br3p, wf1p, bf1):
    full = lambda shape: pl.BlockSpec(shape, lambda i: tuple(0 for _ in shape))
    return pl.pallas_call(
        _relmlp_body,
        grid=(NBLK,),
        in_specs=[
            pl.BlockSpec((1, RB * K, 128), lambda i: (i, 0, 0)),
            pl.BlockSpec((1, RB, 3), lambda i: (i, 0, 0)),
            full((3, 32)), full((3, 32)), full((3, 32)), full((1, 32)),
            full((1, 32)),
            full((32, 128)), full((1, 128)),
            full((128, 128)), full((1, 128)),
            full((128, 128)), full((1, 128)),
        ],
        out_specs=[
            pl.BlockSpec((1, RB, 128), lambda i: (i, 0, 0)),
            pl.BlockSpec((8, 128), lambda i: (0, 0)),
        ],
        out_shape=[
            jax.ShapeDtypeStruct((NBLK, RB, 128), jnp.float32),
            jax.ShapeDtypeStruct((8, 128), jnp.float32),
        ],
        compiler_params=pltpu.CompilerParams(
            dimension_semantics=("arbitrary",)),
    )(g, pos, wa, wb, wc, w9, br1, wr2, br2, wr3p, br3p, wf1p, bf1)


def _fc2_body(h1_ref, sc1_ref, sh1_ref, wf2_ref, bf2_ref, u_ref, acc_ref):
    i = pl.program_id(0)
    t = jax.nn.relu(h1_ref[0] * sc1_ref[...] + sh1_ref[...])
    u = jnp.dot(t.astype(jnp.bfloat16), wf2_ref[...].astype(jnp.bfloat16),
                preferred_element_type=jnp.float32) + bf2_ref[...]
    u_ref[0] = u

    @pl.when(i == 0)
    def _():
        acc_ref[...] = jnp.zeros((8, 128), jnp.float32)

    acc_ref[0:1, :] += jnp.sum(u, axis=0, keepdims=True)
    acc_ref[1:2, :] += jnp.sum(u * u, axis=0, keepdims=True)


def _fc2(h1, sc1, sh1, wf2, bf2):
    full = lambda shape: pl.BlockSpec(shape, lambda i: tuple(0 for _ in shape))
    return pl.pallas_call(
        _fc2_body,
        grid=(NBLK,),
        in_specs=[
            pl.BlockSpec((1, RB, 128), lambda i: (i, 0, 0)),
            full((1, 128)), full((1, 128)), full((128, 128)), full((1, 128)),
        ],
        out_specs=[
            pl.BlockSpec((1, RB, 128), lambda i: (i, 0, 0)),
            pl.BlockSpec((8, 128), lambda i: (0, 0)),
        ],
        out_shape=[
            jax.ShapeDtypeStruct((NBLK, RB, 128), jnp.float32),
            jax.ShapeDtypeStruct((8, 128), jnp.float32),
        ],
        compiler_params=pltpu.CompilerParams(
            dimension_semantics=("arbitrary",)),
    )(h1, sc1, sh1, wf2, bf2)


def _bn2_body(u_ref, sc2_ref, sh2_ref, out_ref):
    out_ref[0] = u_ref[0] * sc2_ref[...] + sh2_ref[...]


def _bn2(u, sc2, sh2):
    full = lambda shape: pl.BlockSpec(shape, lambda i: tuple(0 for _ in shape))
    return pl.pallas_call(
        _bn2_body,
        grid=(NBLK,),
        in_specs=[
            pl.BlockSpec((1, RB, 128), lambda i: (i, 0, 0)),
            full((1, 128)), full((1, 128)),
        ],
        out_specs=pl.BlockSpec((1, RB, 128), lambda i: (i, 0, 0)),
        out_shape=jax.ShapeDtypeStruct((NBLK, RB, 128), jnp.float32),
    )(u, sc2, sh2)


@jax.jit
def kernel(x, pos, Wr1, br1, Wr2, br2, Wr3, br3, Wf1, bf1, g1, be1,
           Wf2, bf2, g2, be2):
    f32 = jnp.float32
    perm = jnp.concatenate(
        [jnp.arange(125, 128), jnp.arange(0, 125)]).astype(jnp.int32)
    wr3p = Wr3[:, perm]
    br3p = br3[perm].reshape(1, 128)
    wf1p = Wf1[perm, :]

    wa = Wr1[0:3]
    wb = Wr1[3:6]
    wc = Wr1[6:9]
    w9 = Wr1[9:10]

    post = pos.transpose(0, 2, 1)
    gidx = _topk(pos, post)
    feat = jnp.concatenate([pos, x], axis=-1).reshape(BZ * N, 128)
    g = _sc_gather(feat, gidx.reshape(NW, NCH, CHUNK))
    g = g.reshape(NBLK, RB * K, 128)

    h1, acc1 = _relmlp(
        g, pos.reshape(NBLK, RB, 3), wa, wb, wc, w9, br1.reshape(1, 32),
        Wr2, br2.reshape(1, 128), wr3p, br3p, wf1p, bf1.reshape(1, 128))

    cnt = jnp.float32(BZ * N)
    m1 = acc1[0:1] / cnt
    v1 = acc1[1:2] / cnt - m1 * m1
    sc1 = g1.reshape(1, 128) / jnp.sqrt(v1 + 1e-5)
    sh1 = be1.reshape(1, 128) - m1 * sc1

    u, acc2 = _fc2(h1, sc1, sh1, Wf2, bf2.reshape(1, 128))
    m2 = acc2[0:1] / cnt
    v2 = acc2[1:2] / cnt - m2 * m2
    sc2 = g2.reshape(1, 128) / jnp.sqrt(v2 + 1e-5)
    sh2 = be2.reshape(1, 128) - m2 * sc2

    out = _bn2(u, sc2, sh2)
    return out.reshape(BZ, N, 128).astype(f32)
